# baseline (speedup 1.0000x reference)
import functools

import jax
import jax.numpy as jnp
from jax.experimental import pallas as pl
from jax.experimental.pallas import tpu as pltpu

_N = 4194304
_COLS = 1024
_ROWS = _N // _COLS
_BROWS = 512
_GRID = _ROWS // _BROWS


def _tc_body(y_ref, s_ref, oy_ref, oys_ref, os_ref):
    i = pl.program_id(0)
    yv = y_ref[...]
    sv = s_ref[...]
    py = jnp.sum(yv)
    pys = jnp.sum(jnp.where(sv == 1, yv, jnp.float32(0.0)))
    ps = jnp.sum(sv.astype(jnp.float32))

    @pl.when(i == 0)
    def _init():
        oy_ref[0, 0] = py
        oys_ref[0, 0] = pys
        os_ref[0, 0] = ps

    @pl.when(i != 0)
    def _acc():
        oy_ref[0, 0] += py
        oys_ref[0, 0] += pys
        os_ref[0, 0] += ps


_tc_reduce = pl.pallas_call(
    _tc_body,
    grid=(_GRID,),
    in_specs=[
        pl.BlockSpec((_BROWS, _COLS), lambda i: (i, 0)),
        pl.BlockSpec((_BROWS, _COLS), lambda i: (i, 0)),
    ],
    out_specs=[
        pl.BlockSpec(memory_space=pltpu.SMEM),
        pl.BlockSpec(memory_space=pltpu.SMEM),
        pl.BlockSpec(memory_space=pltpu.SMEM),
    ],
    out_shape=[
        jax.ShapeDtypeStruct((1, 1), jnp.float32),
        jax.ShapeDtypeStruct((1, 1), jnp.float32),
        jax.ShapeDtypeStruct((1, 1), jnp.float32),
    ],
    compiler_params=pltpu.CompilerParams(
        dimension_semantics=("arbitrary",),
    ),
)


def kernel(y_pred, s):
    y2 = y_pred.reshape(_ROWS, _COLS)
    s2 = s.reshape(_ROWS, _COLS)
    sy, sys_, cnt1 = _tc_reduce(y2, s2)
    sum_y = sy[0, 0]
    sum_ys = sys_[0, 0]
    c1 = cnt1[0, 0]
    c0 = jnp.float32(_N) - c1
    mean1 = sum_ys / c1
    mean0 = (sum_y - sum_ys) / c0
    return jnp.abs(mean0 - mean1)
